# baseline (speedup 1.0000x reference)
import functools

import jax
import jax.numpy as jnp
from jax import lax
from jax.experimental import pallas as pl
from jax.experimental.pallas import tpu as pltpu
from jax.experimental.pallas import tpu_sc as plsc

D = 128
HID = 64

_NC, _NS = 2, 16
_NW = _NC * _NS
_SUB = 96
_TOTCH = 640
_NPAD = _SUB * _TOTCH
_N0, _N1 = 27, 13
_NMAX = max(_N0, _N1)
_TOTCH_PAD = _TOTCH + _NMAX + 5

_BA = 1000
_BC = 1024


def _project_body(x_ref, wproj_ref, w1a_ref, b1a_ref, wh1_ref, bh1_ref,
                  v2_ref, v3_ref, u3_ref, v4a_ref, v4b_ref, o1_ref):
    x = x_ref[...]
    p = jnp.dot(x, wproj_ref[...], preferred_element_type=jnp.float32)
    v2_ref[...] = p[:, 0:128]
    v3_ref[...] = p[:, 128:256]
    u3_ref[...] = p[:, 256:384]
    v4a_ref[...] = p[:, 384:512]
    v4b_ref[...] = p[:, 512:640]
    s1 = jnp.maximum(
        jnp.dot(x, w1a_ref[...], preferred_element_type=jnp.float32)
        + b1a_ref[...], 0.0)
    o1_ref[...] = (jnp.dot(s1, wh1_ref[...], preferred_element_type=jnp.float32)
                   + bh1_ref[...])


def _project(h, wproj, w1a, b1a, wh1, bh1):
    n1 = h.shape[0]
    grid = (n1 // _BA,)
    row = lambda i: (i, 0)
    rep = lambda i: (0, 0)
    return pl.pallas_call(
        _project_body,
        grid=grid,
        in_specs=[
            pl.BlockSpec((_BA, D), row),
            pl.BlockSpec((D, 640), rep),
            pl.BlockSpec((D, HID), rep),
            pl.BlockSpec((1, HID), rep),
            pl.BlockSpec((HID, 3), rep),
            pl.BlockSpec((1, 3), rep),
        ],
        out_specs=[
            pl.BlockSpec((_BA, 128), row),
            pl.BlockSpec((_BA, 128), row),
            pl.BlockSpec((_BA, 128), row),
            pl.BlockSpec((_BA, 128), row),
            pl.BlockSpec((_BA, 128), row),
            pl.BlockSpec((_BA, 3), row),
        ],
        out_shape=[
            jax.ShapeDtypeStruct((n1, 128), jnp.float32),
            jax.ShapeDtypeStruct((n1, 128), jnp.float32),
            jax.ShapeDtypeStruct((n1, 128), jnp.float32),
            jax.ShapeDtypeStruct((n1, 128), jnp.float32),
            jax.ShapeDtypeStruct((n1, 128), jnp.float32),
            jax.ShapeDtypeStruct((n1, 3), jnp.float32),
        ],
    )(h, wproj, w1a, b1a, wh1, bh1)


_GATHER_CFG = {
    2: ((128, 128), (0, 0), 1),
    3: ((128, 128, 128), (0, 1, 0), 2),
    4: ((128, 128, 128, 128), (0, 1, 1, 0), 2),
}


_PHASES = ((2, (0, 0)), (3, (1, 2, 1)), (4, (3, 4, 4, 3)))


@functools.lru_cache(maxsize=None)
def _make_gather_all():
    mesh = plsc.VectorSubcoreMesh(core_axis_name="c", subcore_axis_name="s",
                                  num_cores=_NC, num_subcores=_NS)
    out_type = tuple(
        jax.ShapeDtypeStruct((_NPAD, 128), jnp.float32) for _ in range(9))
    scratch = (
        [pltpu.VMEM((_NMAX * _SUB,), jnp.int32) for _ in range(4)]
        + [pltpu.VMEM((2, _SUB, 128), jnp.float32) for _ in range(4)]
        + [pltpu.SemaphoreType.DMA for _ in range(4)]
        + [pltpu.SemaphoreType.DMA for _ in range(4)]
    )

    def body(idx_hbm, *refs):
        tabs = refs[:5]
        outs = refs[5:14]
        idxv = refs[14:18]
        gbuf = refs[18:22]
        gsem = refs[22:26]
        osem = refs[26:30]
        cc = lax.axis_index("c")
        ss = lax.axis_index("s")
        nch = jnp.where(cc == 0, _N0, _N1)
        cb = jnp.where(cc == 0, ss * _N0, _NS * _N0 + ss * _N1)

        obase = 0
        lvlbase = 0
        for level, tmap in _PHASES:
            louts = outs[obase:obase + level]
            for c in range(level):
                off = pl.multiple_of(
                    (lvlbase + c * _TOTCH_PAD + cb) * _SUB, _SUB)
                pltpu.sync_copy(idx_hbm.at[pl.ds(off, _NMAX * _SUB)], idxv[c])

            def gather_cp(c, k, slot, tmap=tmap):
                koff = pl.multiple_of(k * _SUB, _SUB)
                return pltpu.make_async_copy(
                    tabs[tmap[c]].at[idxv[c].at[pl.ds(koff, _SUB)]],
                    gbuf[c].at[slot], gsem[c])

            def out_cp(c, k, slot, louts=louts):
                roff = pl.multiple_of((cb + k) * _SUB, _SUB)
                return pltpu.make_async_copy(
                    gbuf[c].at[slot],
                    louts[c].at[pl.ds(roff, _SUB)], osem[c])

            for c in range(level):
                gather_cp(c, 0, 0).start()

            def step(k, _, level=level, gather_cp=gather_cp, out_cp=out_cp):
                par = lax.rem(k, 2)
                for c in range(level):
                    gather_cp(c, k, par).wait()

                @pl.when(k > 0)
                def _():
                    for c in range(level):
                        out_cp(c, k - 1, 1 - par).wait()

                @pl.when(k + 1 < nch)
                def _():
                    for c in range(level):
                        gather_cp(c, k + 1, 1 - par).start()
                for c in range(level):
                    out_cp(c, k, par).start()
                return None

            lax.fori_loop(0, nch, step, None)
            last = nch - 1
            for c in range(level):
                out_cp(c, last, lax.rem(last, 2)).wait()
            obase += level
            lvlbase += level * _TOTCH_PAD

    return pl.kernel(body, out_type=out_type, mesh=mesh,
                     scratch_types=scratch, name="sc_gather_all")


def _gather_all(idx, v2, v3, u3, v4a, v4b):
    return _make_gather_all()(idx, v2, v3, u3, v4a, v4b)


def _post_body(g20_ref, g21_ref, g30_ref, g31_ref, g32_ref,
               g40_ref, g41_ref, g42_ref, g43_ref,
               b2_ref, w2_ref, c2_ref, b3_ref, w3_ref, c3_ref,
               b4_ref, w4_ref, c4_ref,
               o2_ref, o3_ref, o4_ref):
    def head(af, ab, b_ref, w_ref, c_ref):
        b = b_ref[...]
        m = jnp.maximum(af + b, 0.0) + jnp.maximum(ab + b, 0.0)
        return (jnp.dot(m, w_ref[...], preferred_element_type=jnp.float32)
                + c_ref[...])

    g20 = g20_ref[...]
    g21 = g21_ref[...]
    o2_ref[...] = head(g20[:, :64] + g21[:, 64:], g20[:, 64:] + g21[:, :64],
                       b2_ref, w2_ref, c2_ref)
    g30 = g30_ref[...]
    g31 = g31_ref[...]
    g32 = g32_ref[...]
    o3_ref[...] = head(g30[:, :64] + g31[:, :64] + g32[:, 64:],
                       g30[:, 64:] + g31[:, 64:] + g32[:, :64],
                       b3_ref, w3_ref, c3_ref)
    g40 = g40_ref[...]
    g41 = g41_ref[...]
    g42 = g42_ref[...]
    g43 = g43_ref[...]
    o4_ref[...] = head(g40[:, :64] + g41[:, :64] + g42[:, 64:] + g43[:, 64:],
                       g40[:, 64:] + g41[:, 64:] + g42[:, :64] + g43[:, :64],
                       b4_ref, w4_ref, c4_ref)


def _post(gs, wcs):
    grid = (_NPAD // _BC,)
    row = lambda i: (i, 0)
    rep = lambda i: (0, 0)
    g_specs = [pl.BlockSpec((_BC, g.shape[1]), row) for g in gs]
    w_specs = []
    for b, w, c in wcs:
        w_specs += [pl.BlockSpec((1, HID), rep),
                    pl.BlockSpec((HID, 2), rep),
                    pl.BlockSpec((1, 2), rep)]
    flat_w = [a for bwc in wcs for a in bwc]
    return pl.pallas_call(
        _post_body,
        grid=grid,
        in_specs=g_specs + w_specs,
        out_specs=[pl.BlockSpec((_BC, 2), row)] * 3,
        out_shape=[jax.ShapeDtypeStruct((_NPAD, 2), jnp.float32)] * 3,
    )(*gs, *flat_w)


def kernel(h, idx2, idx3, idx4, params):
    p = params
    w2, w3, w4 = p['W2a'], p['W3a'], p['W4a']
    wproj = jnp.concatenate([
        w2[0:128], w2[128:256],
        w3[0:128], w3[256:384],
        w3[128:256], w3[128:256],
        w4[0:128], w4[384:512],
        w4[128:256], w4[256:384],
    ], axis=1)
    wh1 = jnp.concatenate([p['W1_sigma'], p['W1_epsilon'], p['W1_q']], axis=1)
    bh1 = jnp.stack([p['b1_sigma'], p['b1_epsilon'], p['b1_q']], axis=1)
    v2, v3, u3, v4a, v4b, o1 = _project(
        h, wproj, p['W1a'], p['b1a'].reshape(1, HID), wh1, bh1.reshape(1, 3))

    def prep_idx(idx, level):
        it = jnp.pad(idx.T, ((0, 0), (0, _TOTCH_PAD * _SUB - idx.shape[0])))
        return it.reshape(level * _TOTCH_PAD * _SUB)

    idx_all = jnp.concatenate(
        [prep_idx(idx2, 2), prep_idx(idx3, 3), prep_idx(idx4, 4)])
    gs = _gather_all(idx_all, v2, v3, u3, v4a, v4b)

    wcs = []
    for L in (2, 3, 4):
        wcs.append((p['b%da' % L].reshape(1, HID),
                    jnp.concatenate([p['W%d_k' % L], p['W%d_eq' % L]], axis=1),
                    jnp.stack([p['b%d_k' % L], p['b%d_eq' % L]],
                              axis=1).reshape(1, 2)))
    o2, o3, o4 = _post(list(gs), wcs)

    n2, n3, n4 = idx2.shape[0], idx3.shape[0], idx4.shape[0]
    return (o2[:n2, 0:1], o2[:n2, 1:2],
            o3[:n3, 0:1], o3[:n3, 1:2],
            o4[:n4, 0:1], o4[:n4, 1:2],
            o1[:, 0:1], o1[:, 1:2], o1[:, 2:3])


# speedup vs baseline: 1.0072x; 1.0072x over previous
import functools

import jax
import jax.numpy as jnp
from jax import lax
from jax.experimental import pallas as pl
from jax.experimental.pallas import tpu as pltpu
from jax.experimental.pallas import tpu_sc as plsc

D = 128
HID = 64

_NC, _NS = 2, 16
_NW = _NC * _NS
_SUB = 96
_TOTCH = 640
_NPAD = _SUB * _TOTCH
_N0, _N1 = 27, 13
_NMAX = max(_N0, _N1)
_TOTCH_PAD = _TOTCH + _NMAX + 5

_BA = 1000
_BC = 1024


def _project_body(x_ref, wproj_ref, w1a_ref, b1a_ref, wh1_ref, bh1_ref,
                  v2_ref, v3_ref, u3_ref, v4a_ref, v4b_ref, o1_ref):
    x = x_ref[...]
    p = jnp.dot(x, wproj_ref[...], preferred_element_type=jnp.float32)
    v2_ref[...] = p[:, 0:128]
    v3_ref[...] = p[:, 128:256]
    u3_ref[...] = p[:, 256:384]
    v4a_ref[...] = p[:, 384:512]
    v4b_ref[...] = p[:, 512:640]
    s1 = jnp.maximum(
        jnp.dot(x, w1a_ref[...], preferred_element_type=jnp.float32)
        + b1a_ref[...], 0.0)
    o1_ref[...] = (jnp.dot(s1, wh1_ref[...], preferred_element_type=jnp.float32)
                   + bh1_ref[...])


def _project(h, wproj, w1a, b1a, wh1, bh1):
    n1 = h.shape[0]
    grid = (n1 // _BA,)
    row = lambda i: (i, 0)
    rep = lambda i: (0, 0)
    return pl.pallas_call(
        _project_body,
        grid=grid,
        in_specs=[
            pl.BlockSpec((_BA, D), row),
            pl.BlockSpec((D, 640), rep),
            pl.BlockSpec((D, HID), rep),
            pl.BlockSpec((1, HID), rep),
            pl.BlockSpec((HID, 3), rep),
            pl.BlockSpec((1, 3), rep),
        ],
        out_specs=[
            pl.BlockSpec((_BA, 128), row),
            pl.BlockSpec((_BA, 128), row),
            pl.BlockSpec((_BA, 128), row),
            pl.BlockSpec((_BA, 128), row),
            pl.BlockSpec((_BA, 128), row),
            pl.BlockSpec((_BA, 3), row),
        ],
        out_shape=[
            jax.ShapeDtypeStruct((n1, 128), jnp.float32),
            jax.ShapeDtypeStruct((n1, 128), jnp.float32),
            jax.ShapeDtypeStruct((n1, 128), jnp.float32),
            jax.ShapeDtypeStruct((n1, 128), jnp.float32),
            jax.ShapeDtypeStruct((n1, 128), jnp.float32),
            jax.ShapeDtypeStruct((n1, 3), jnp.float32),
        ],
    )(h, wproj, w1a, b1a, wh1, bh1)


_GATHER_CFG = {
    2: ((128, 128), (0, 0), 1),
    3: ((128, 128, 128), (0, 1, 0), 2),
    4: ((128, 128, 128, 128), (0, 1, 1, 0), 2),
}


_PHASES = ((2, (0, 0)), (3, (1, 2, 1)), (4, (3, 4, 4, 3)))


@functools.lru_cache(maxsize=None)
def _make_gather_all():
    mesh = plsc.VectorSubcoreMesh(core_axis_name="c", subcore_axis_name="s",
                                  num_cores=_NC, num_subcores=_NS)
    out_type = tuple(
        jax.ShapeDtypeStruct((_NPAD, 128), jnp.float32) for _ in range(9))
    scratch = (
        [pltpu.VMEM((_NMAX * _SUB,), jnp.int32) for _ in range(9)]
        + [pltpu.VMEM((2, _SUB, 128), jnp.float32) for _ in range(4)]
        + [pltpu.SemaphoreType.DMA for _ in range(4)]
        + [pltpu.SemaphoreType.DMA for _ in range(4)]
        + [pltpu.SemaphoreType.DMA]
    )

    def body(idx_hbm, *refs):
        tabs = refs[:5]
        outs = refs[5:14]
        idxv = refs[14:23]
        gbuf = refs[23:27]
        gsem = refs[27:31]
        osem = refs[31:35]
        isem = refs[35]
        cc = lax.axis_index("c")
        ss = lax.axis_index("s")
        nch = jnp.where(cc == 0, _N0, _N1)
        cb = jnp.where(cc == 0, ss * _N0, _NS * _N0 + ss * _N1)

        icps = []
        for g in range(9):
            off = pl.multiple_of((g * _TOTCH_PAD + cb) * _SUB, _SUB)
            icps.append(pltpu.make_async_copy(
                idx_hbm.at[pl.ds(off, _NMAX * _SUB)], idxv[g], isem))
            icps[-1].start()
        for cp in icps:
            cp.wait()

        obase = 0
        for level, tmap in _PHASES:
            louts = outs[obase:obase + level]
            lidx = idxv[obase:obase + level]

            def gather_cp(c, k, slot, tmap=tmap, lidx=lidx):
                koff = pl.multiple_of(k * _SUB, _SUB)
                return pltpu.make_async_copy(
                    tabs[tmap[c]].at[lidx[c].at[pl.ds(koff, _SUB)]],
                    gbuf[c].at[slot], gsem[c])

            def out_cp(c, k, slot, louts=louts):
                roff = pl.multiple_of((cb + k) * _SUB, _SUB)
                return pltpu.make_async_copy(
                    gbuf[c].at[slot],
                    louts[c].at[pl.ds(roff, _SUB)], osem[c])

            for c in range(level):
                gather_cp(c, 0, 0).start()

            def step(k, _, level=level, gather_cp=gather_cp, out_cp=out_cp):
                par = lax.rem(k, 2)
                for c in range(level):
                    gather_cp(c, k, par).wait()

                @pl.when(k > 0)
                def _():
                    for c in range(level):
                        out_cp(c, k - 1, 1 - par).wait()

                @pl.when(k + 1 < nch)
                def _():
                    for c in range(level):
                        gather_cp(c, k + 1, 1 - par).start()
                for c in range(level):
                    out_cp(c, k, par).start()
                return None

            lax.fori_loop(0, nch, step, None)
            last = nch - 1
            for c in range(level):
                out_cp(c, last, lax.rem(last, 2)).wait()
            obase += level

    return pl.kernel(body, out_type=out_type, mesh=mesh,
                     scratch_types=scratch, name="sc_gather_all")


def _gather_all(idx, v2, v3, u3, v4a, v4b):
    return _make_gather_all()(idx, v2, v3, u3, v4a, v4b)


def _post_body(g20_ref, g21_ref, g30_ref, g31_ref, g32_ref,
               g40_ref, g41_ref, g42_ref, g43_ref,
               b2_ref, w2_ref, c2_ref, b3_ref, w3_ref, c3_ref,
               b4_ref, w4_ref, c4_ref,
               o2_ref, o3_ref, o4_ref):
    def head(af, ab, b_ref, w_ref, c_ref):
        b = b_ref[...]
        m = jnp.maximum(af + b, 0.0) + jnp.maximum(ab + b, 0.0)
        return (jnp.dot(m, w_ref[...], preferred_element_type=jnp.float32)
                + c_ref[...])

    g20 = g20_ref[...]
    g21 = g21_ref[...]
    o2_ref[...] = head(g20[:, :64] + g21[:, 64:], g20[:, 64:] + g21[:, :64],
                       b2_ref, w2_ref, c2_ref)
    g30 = g30_ref[...]
    g31 = g31_ref[...]
    g32 = g32_ref[...]
    o3_ref[...] = head(g30[:, :64] + g31[:, :64] + g32[:, 64:],
                       g30[:, 64:] + g31[:, 64:] + g32[:, :64],
                       b3_ref, w3_ref, c3_ref)
    g40 = g40_ref[...]
    g41 = g41_ref[...]
    g42 = g42_ref[...]
    g43 = g43_ref[...]
    o4_ref[...] = head(g40[:, :64] + g41[:, :64] + g42[:, 64:] + g43[:, 64:],
                       g40[:, 64:] + g41[:, 64:] + g42[:, :64] + g43[:, :64],
                       b4_ref, w4_ref, c4_ref)


def _post(gs, wcs):
    grid = (_NPAD // _BC,)
    row = lambda i: (i, 0)
    rep = lambda i: (0, 0)
    g_specs = [pl.BlockSpec((_BC, g.shape[1]), row) for g in gs]
    w_specs = []
    for b, w, c in wcs:
        w_specs += [pl.BlockSpec((1, HID), rep),
                    pl.BlockSpec((HID, 2), rep),
                    pl.BlockSpec((1, 2), rep)]
    flat_w = [a for bwc in wcs for a in bwc]
    return pl.pallas_call(
        _post_body,
        grid=grid,
        in_specs=g_specs + w_specs,
        out_specs=[pl.BlockSpec((_BC, 2), row)] * 3,
        out_shape=[jax.ShapeDtypeStruct((_NPAD, 2), jnp.float32)] * 3,
    )(*gs, *flat_w)


def kernel(h, idx2, idx3, idx4, params):
    p = params
    w2, w3, w4 = p['W2a'], p['W3a'], p['W4a']
    wproj = jnp.concatenate([
        w2[0:128], w2[128:256],
        w3[0:128], w3[256:384],
        w3[128:256], w3[128:256],
        w4[0:128], w4[384:512],
        w4[128:256], w4[256:384],
    ], axis=1)
    wh1 = jnp.concatenate([p['W1_sigma'], p['W1_epsilon'], p['W1_q']], axis=1)
    bh1 = jnp.stack([p['b1_sigma'], p['b1_epsilon'], p['b1_q']], axis=1)
    v2, v3, u3, v4a, v4b, o1 = _project(
        h, wproj, p['W1a'], p['b1a'].reshape(1, HID), wh1, bh1.reshape(1, 3))

    def prep_idx(idx, level):
        it = jnp.pad(idx.T, ((0, 0), (0, _TOTCH_PAD * _SUB - idx.shape[0])))
        return it.reshape(level * _TOTCH_PAD * _SUB)

    idx_all = jnp.concatenate(
        [prep_idx(idx2, 2), prep_idx(idx3, 3), prep_idx(idx4, 4)])
    gs = _gather_all(idx_all, v2, v3, u3, v4a, v4b)

    wcs = []
    for L in (2, 3, 4):
        wcs.append((p['b%da' % L].reshape(1, HID),
                    jnp.concatenate([p['W%d_k' % L], p['W%d_eq' % L]], axis=1),
                    jnp.stack([p['b%d_k' % L], p['b%d_eq' % L]],
                              axis=1).reshape(1, 2)))
    o2, o3, o4 = _post(list(gs), wcs)

    n2, n3, n4 = idx2.shape[0], idx3.shape[0], idx4.shape[0]
    return (o2[:n2, 0:1], o2[:n2, 1:2],
            o3[:n3, 0:1], o3[:n3, 1:2],
            o4[:n4, 0:1], o4[:n4, 1:2],
            o1[:, 0:1], o1[:, 1:2], o1[:, 2:3])


# speedup vs baseline: 1.0318x; 1.0244x over previous
import functools

import jax
import jax.numpy as jnp
from jax import lax
from jax.experimental import pallas as pl
from jax.experimental.pallas import tpu as pltpu
from jax.experimental.pallas import tpu_sc as plsc

D = 128
HID = 64

_NC, _NS = 2, 16
_NW = _NC * _NS
_SUB = 96
_TOTCH = 640
_NPAD = _SUB * _TOTCH
_N0, _N1 = 27, 13
_NMAX = max(_N0, _N1)
_TOTCH_PAD = _TOTCH + _NMAX + 5

_BA = 1000
_BC = 1024


def _project_body(x_ref, wproj_ref, w1a_ref, b1a_ref, wh1_ref, bh1_ref,
                  v2_ref, v3_ref, u3_ref, v4a_ref, v4b_ref, o1_ref):
    x = x_ref[...]
    p = jnp.dot(x, wproj_ref[...], preferred_element_type=jnp.float32)
    v2_ref[...] = p[:, 0:128]
    v3_ref[...] = p[:, 128:256]
    u3_ref[...] = p[:, 256:384]
    v4a_ref[...] = p[:, 384:512]
    v4b_ref[...] = p[:, 512:640]
    s1 = jnp.maximum(
        jnp.dot(x, w1a_ref[...], preferred_element_type=jnp.float32)
        + b1a_ref[...], 0.0)
    o1_ref[...] = (jnp.dot(s1, wh1_ref[...], preferred_element_type=jnp.float32)
                   + bh1_ref[...])


def _project(h, wproj, w1a, b1a, wh1, bh1):
    n1 = h.shape[0]
    grid = (n1 // _BA,)
    row = lambda i: (i, 0)
    rep = lambda i: (0, 0)
    return pl.pallas_call(
        _project_body,
        grid=grid,
        in_specs=[
            pl.BlockSpec((_BA, D), row),
            pl.BlockSpec((D, 640), rep),
            pl.BlockSpec((D, HID), rep),
            pl.BlockSpec((1, HID), rep),
            pl.BlockSpec((HID, 3), rep),
            pl.BlockSpec((1, 3), rep),
        ],
        out_specs=[
            pl.BlockSpec((_BA, 128), row),
            pl.BlockSpec((_BA, 128), row),
            pl.BlockSpec((_BA, 128), row),
            pl.BlockSpec((_BA, 128), row),
            pl.BlockSpec((_BA, 128), row),
            pl.BlockSpec((_BA, 3), row),
        ],
        out_shape=[
            jax.ShapeDtypeStruct((n1, 128), jnp.float32),
            jax.ShapeDtypeStruct((n1, 128), jnp.float32),
            jax.ShapeDtypeStruct((n1, 128), jnp.float32),
            jax.ShapeDtypeStruct((n1, 128), jnp.float32),
            jax.ShapeDtypeStruct((n1, 128), jnp.float32),
            jax.ShapeDtypeStruct((n1, 3), jnp.float32),
        ],
    )(h, wproj, w1a, b1a, wh1, bh1)


_GATHER_CFG = {
    2: ((128, 128), (0, 0), 1),
    3: ((128, 128, 128), (0, 1, 0), 2),
    4: ((128, 128, 128, 128), (0, 1, 1, 0), 2),
}


@functools.lru_cache(maxsize=None)
def _make_gather(level):
    widths, tab_of_col, n_tabs = _GATHER_CFG[level]
    mesh = plsc.VectorSubcoreMesh(core_axis_name="c", subcore_axis_name="s",
                                  num_cores=_NC, num_subcores=_NS)
    out_type = tuple(
        jax.ShapeDtypeStruct((_NPAD, widths[c]), jnp.float32)
        for c in range(level))
    scratch = (
        [pltpu.VMEM((_NMAX * _SUB,), jnp.int32) for _ in range(level)]
        + [pltpu.VMEM((2, _SUB, widths[c]), jnp.float32) for c in range(level)]
        + [pltpu.SemaphoreType.DMA for _ in range(level)]
        + [pltpu.SemaphoreType.DMA for _ in range(level)]
    )

    def body(idx_hbm, *refs):
        tabs = refs[:n_tabs]
        outs = refs[n_tabs:n_tabs + level]
        idxv = refs[n_tabs + level:n_tabs + 2 * level]
        gbuf = refs[n_tabs + 2 * level:n_tabs + 3 * level]
        gsem = refs[n_tabs + 3 * level:n_tabs + 4 * level]
        osem = refs[n_tabs + 4 * level:n_tabs + 5 * level]
        cc = lax.axis_index("c")
        ss = lax.axis_index("s")
        nch = jnp.where(cc == 0, _N0, _N1)
        cb = jnp.where(cc == 0, ss * _N0, _NS * _N0 + ss * _N1)
        for c in range(level):
            off = pl.multiple_of((c * _TOTCH_PAD + cb) * _SUB, _SUB)
            pltpu.sync_copy(idx_hbm.at[pl.ds(off, _NMAX * _SUB)], idxv[c])

        def gather_cp(c, k, slot):
            koff = pl.multiple_of(k * _SUB, _SUB)
            return pltpu.make_async_copy(
                tabs[tab_of_col[c]].at[idxv[c].at[pl.ds(koff, _SUB)]],
                gbuf[c].at[slot], gsem[c])

        def out_cp(c, k, slot):
            roff = pl.multiple_of((cb + k) * _SUB, _SUB)
            return pltpu.make_async_copy(
                gbuf[c].at[slot],
                outs[c].at[pl.ds(roff, _SUB)], osem[c])

        for c in range(level):
            gather_cp(c, 0, 0).start()

        def step(k, _):
            par = lax.rem(k, 2)
            for c in range(level):
                gather_cp(c, k, par).wait()

            @pl.when(k > 0)
            def _():
                for c in range(level):
                    out_cp(c, k - 1, 1 - par).wait()

            @pl.when(k + 1 < nch)
            def _():
                for c in range(level):
                    gather_cp(c, k + 1, 1 - par).start()
            for c in range(level):
                out_cp(c, k, par).start()
            return None

        lax.fori_loop(0, nch, step, None)
        last = nch - 1
        for c in range(level):
            out_cp(c, last, lax.rem(last, 2)).wait()

    return pl.kernel(body, out_type=out_type, mesh=mesh,
                     scratch_types=scratch, name="sc_gather_l%d" % level)


def _gather2(idx, v2):
    return _make_gather(2)(idx, v2)


def _gather3(idx, v3, u3):
    return _make_gather(3)(idx, v3, u3)


def _gather4(idx, v4a, v4b):
    return _make_gather(4)(idx, v4a, v4b)


def _post_body(g20_ref, g21_ref, g30_ref, g31_ref, g32_ref,
               g40_ref, g41_ref, g42_ref, g43_ref,
               b2_ref, w2_ref, c2_ref, b3_ref, w3_ref, c3_ref,
               b4_ref, w4_ref, c4_ref,
               o2_ref, o3_ref, o4_ref):
    def head(af, ab, b_ref, w_ref, c_ref):
        b = b_ref[...]
        m = jnp.maximum(af + b, 0.0) + jnp.maximum(ab + b, 0.0)
        return (jnp.dot(m, w_ref[...], preferred_element_type=jnp.float32)
                + c_ref[...])

    g20 = g20_ref[...]
    g21 = g21_ref[...]
    o2_ref[...] = head(g20[:, :64] + g21[:, 64:], g20[:, 64:] + g21[:, :64],
                       b2_ref, w2_ref, c2_ref)
    g30 = g30_ref[...]
    g31 = g31_ref[...]
    g32 = g32_ref[...]
    o3_ref[...] = head(g30[:, :64] + g31[:, :64] + g32[:, 64:],
                       g30[:, 64:] + g31[:, 64:] + g32[:, :64],
                       b3_ref, w3_ref, c3_ref)
    g40 = g40_ref[...]
    g41 = g41_ref[...]
    g42 = g42_ref[...]
    g43 = g43_ref[...]
    o4_ref[...] = head(g40[:, :64] + g41[:, :64] + g42[:, 64:] + g43[:, 64:],
                       g40[:, 64:] + g41[:, 64:] + g42[:, :64] + g43[:, :64],
                       b4_ref, w4_ref, c4_ref)


def _post(gs, wcs):
    grid = (_NPAD // _BC,)
    row = lambda i: (i, 0)
    rep = lambda i: (0, 0)
    g_specs = [pl.BlockSpec((_BC, g.shape[1]), row) for g in gs]
    w_specs = []
    for b, w, c in wcs:
        w_specs += [pl.BlockSpec((1, HID), rep),
                    pl.BlockSpec((HID, 2), rep),
                    pl.BlockSpec((1, 2), rep)]
    flat_w = [a for bwc in wcs for a in bwc]
    return pl.pallas_call(
        _post_body,
        grid=grid,
        in_specs=g_specs + w_specs,
        out_specs=[pl.BlockSpec((_BC, 2), row)] * 3,
        out_shape=[jax.ShapeDtypeStruct((_NPAD, 2), jnp.float32)] * 3,
    )(*gs, *flat_w)


def kernel(h, idx2, idx3, idx4, params):
    p = params
    w2, w3, w4 = p['W2a'], p['W3a'], p['W4a']
    wproj = jnp.concatenate([
        w2[0:128], w2[128:256],
        w3[0:128], w3[256:384],
        w3[128:256], w3[128:256],
        w4[0:128], w4[384:512],
        w4[128:256], w4[256:384],
    ], axis=1)
    wh1 = jnp.concatenate([p['W1_sigma'], p['W1_epsilon'], p['W1_q']], axis=1)
    bh1 = jnp.stack([p['b1_sigma'], p['b1_epsilon'], p['b1_q']], axis=1)
    v2, v3, u3, v4a, v4b, o1 = _project(
        h, wproj, p['W1a'], p['b1a'].reshape(1, HID), wh1, bh1.reshape(1, 3))

    def prep_idx(idx, level):
        it = jnp.pad(idx.T, ((0, 0), (0, _TOTCH_PAD * _SUB - idx.shape[0])))
        return it.reshape(level * _TOTCH_PAD * _SUB)

    g2 = _gather2(prep_idx(idx2, 2), v2)
    g3 = _gather3(prep_idx(idx3, 3), v3, u3)
    g4 = _gather4(prep_idx(idx4, 4), v4a, v4b)
    gs = list(g2) + list(g3) + list(g4)

    wcs = []
    for L in (2, 3, 4):
        wcs.append((p['b%da' % L].reshape(1, HID),
                    jnp.concatenate([p['W%d_k' % L], p['W%d_eq' % L]], axis=1),
                    jnp.stack([p['b%d_k' % L], p['b%d_eq' % L]],
                              axis=1).reshape(1, 2)))
    o2, o3, o4 = _post(list(gs), wcs)

    n2, n3, n4 = idx2.shape[0], idx3.shape[0], idx4.shape[0]
    return (o2[:n2, 0:1], o2[:n2, 1:2],
            o3[:n3, 0:1], o3[:n3, 1:2],
            o4[:n4, 0:1], o4[:n4, 1:2],
            o1[:, 0:1], o1[:, 1:2], o1[:, 2:3])
